# baseline (speedup 1.0000x reference)
import functools

import jax
import jax.numpy as jnp
from jax import lax
from jax.experimental import pallas as pl
from jax.experimental.pallas import tpu as pltpu
from jax.experimental.pallas import tpu_sc as plsc

VOCAB = 100000
DIM = 32
B = 1024
L = 200

_NC = 2
_NS = 16
_NW = _NC * _NS
_SPW = B // _NW


def _pool_body(xt_hbm, emb_hbm, out_hbm, idxt_v, acc_v, sem):
    wid = lax.axis_index("s") * _NC + lax.axis_index("c")
    base = wid * _SPW
    pltpu.sync_copy(xt_hbm.at[:, pl.ds(base, _SPW)], idxt_v)
    z = jnp.zeros((16,), jnp.float32)

    def zrow(s, carry):
        acc_v[s, pl.ds(0, 16)] = z
        acc_v[s, pl.ds(16, 16)] = z
        return carry

    lax.fori_loop(0, _SPW, zrow, 0)

    def gadd(r, carry):
        pltpu.async_copy(emb_hbm.at[idxt_v.at[r]], acc_v, sem, add=True).wait()
        return carry

    lax.fori_loop(0, L, gadd, 0)
    pltpu.sync_copy(acc_v, out_hbm.at[pl.ds(base, _SPW), :])


def _unused_pool_body(x_hbm, emb_hbm, out_hbm, idx_v, rows_v, acc_v, sem_a, sem_b):
    wid = lax.axis_index("s") * _NC + lax.axis_index("c")
    base = wid * _SPW
    pltpu.sync_copy(x_hbm.at[pl.ds(base, _SPW), :], idx_v)

    def issue(s, buf, sem):
        pltpu.async_copy(
            emb_hbm.at[idx_v.at[s, pl.ds(0, 128)]],
            rows_v.at[buf, pl.ds(0, 128)], sem)
        pltpu.async_copy(
            emb_hbm.at[idx_v.at[s, pl.ds(128, L - 128)]],
            rows_v.at[buf, pl.ds(128, L - 128)], sem)

    def drain(buf, sem):
        pltpu.make_async_copy(
            emb_hbm.at[idx_v.at[0, pl.ds(0, 128)]],
            rows_v.at[buf, pl.ds(0, 128)], sem).wait()
        pltpu.make_async_copy(
            emb_hbm.at[idx_v.at[0, pl.ds(0, L - 128)]],
            rows_v.at[buf, pl.ds(128, L - 128)], sem).wait()

    def accum(buf, s):
        rows = rows_v.at[buf]

        def step(r, accs):
            a0, a1, a2, a3 = accs
            q = r * 4
            a0 = a0 + rows[q, pl.ds(0, 16)]
            a1 = a1 + rows[q, pl.ds(16, 16)]
            a2 = a2 + rows[q + 1, pl.ds(0, 16)]
            a3 = a3 + rows[q + 1, pl.ds(16, 16)]
            a0 = a0 + rows[q + 2, pl.ds(0, 16)]
            a1 = a1 + rows[q + 2, pl.ds(16, 16)]
            a2 = a2 + rows[q + 3, pl.ds(0, 16)]
            a3 = a3 + rows[q + 3, pl.ds(16, 16)]
            return a0, a1, a2, a3

        z = jnp.zeros((16,), jnp.float32)
        a0, a1, a2, a3 = lax.fori_loop(0, L // 4, step, (z, z, z, z))
        acc_v[s, pl.ds(0, 16)] = a0 + a2
        acc_v[s, pl.ds(16, 16)] = a1 + a3

    issue(0, 0, sem_a)

    def pair(p, carry):
        s = p * 2
        issue(s + 1, 1, sem_b)
        drain(0, sem_a)
        accum(0, s)

        @pl.when(p + 1 < _SPW // 2)
        def _():
            issue(s + 2, 0, sem_a)

        drain(1, sem_b)
        accum(1, s + 1)
        return carry

    lax.fori_loop(0, _SPW // 2, pair, 0)
    pltpu.sync_copy(acc_v, out_hbm.at[pl.ds(base, _SPW), :])


_pool = pl.kernel(
    _pool_body,
    out_type=jax.ShapeDtypeStruct((B, DIM), jnp.float32),
    mesh=plsc.VectorSubcoreMesh(core_axis_name="c", subcore_axis_name="s"),
    scratch_types=[
        pltpu.VMEM((L, _SPW), jnp.int32),
        pltpu.VMEM((_SPW, DIM), jnp.float32),
        pltpu.SemaphoreType.DMA,
    ],
    compiler_params=pltpu.CompilerParams(use_tc_tiling_on_sc=False),
)


_VBLK = 2048
_NV = (VOCAB + _VBLK - 1) // _VBLK


def _mmT_body(p_ref, wt_ref, b_ref, o_ref):
    p = p_ref[...] * (1.0 / L)
    acc = lax.dot_general(
        wt_ref[...], p, (((0,), (1,)), ((), ())),
        preferred_element_type=jnp.float32)
    ones = jnp.ones((1, B), jnp.float32)
    bias = lax.dot_general(
        b_ref[...], ones, (((0,), (0,)), ((), ())),
        preferred_element_type=jnp.float32)
    o_ref[...] = acc + bias


def _matmul_t(pooled, WT, b2d):
    return pl.pallas_call(
        _mmT_body,
        grid=(_NV,),
        in_specs=[
            pl.BlockSpec((B, DIM), lambda i: (0, 0)),
            pl.BlockSpec((DIM, _VBLK), lambda i: (0, i)),
            pl.BlockSpec((1, _VBLK), lambda i: (0, i)),
        ],
        out_specs=pl.BlockSpec((_VBLK, B), lambda i: (i, 0)),
        out_shape=jax.ShapeDtypeStruct((VOCAB, B), jnp.float32),
    )(pooled, WT, b2d)


@jax.jit
def _impl(x, emb, W, b):
    sums = _pool(x.T, emb)
    logits_t = _matmul_t(sums, W.T, b.reshape(1, -1))
    return logits_t.T


def kernel(x, emb, W, b):
    return _impl(x, emb, W, b)


# speedup vs baseline: 1.4551x; 1.4551x over previous
import functools

import jax
import jax.numpy as jnp
from jax import lax
from jax.experimental import pallas as pl
from jax.experimental.pallas import tpu as pltpu
from jax.experimental.pallas import tpu_sc as plsc

VOCAB = 100000
DIM = 32
B = 1024
L = 200

_NC = 2
_NS = 16
_NW = _NC * _NS
_SPW = B // _NW


_NBUF = 8


def _pool_body(xt_hbm, emb_hbm, out_hbm, idxt_v, acc_v, accf_v, *sems):
    wid = lax.axis_index("s") * _NC + lax.axis_index("c")
    base = wid * _SPW
    pltpu.sync_copy(xt_hbm.at[:, pl.ds(base, _SPW)], idxt_v)
    z = jnp.zeros((16,), jnp.float32)

    for k in range(_NBUF):
        def zrow(s, carry, k=k):
            acc_v[k, s, pl.ds(0, 16)] = z
            acc_v[k, s, pl.ds(16, 16)] = z
            return carry
        lax.fori_loop(0, _SPW, zrow, 0)

    def issue(r, k):
        pltpu.async_copy(emb_hbm.at[idxt_v.at[r]], acc_v.at[k], sems[k],
                         add=True)

    def wait(k):
        pltpu.make_async_copy(emb_hbm.at[idxt_v.at[0]], acc_v.at[k],
                              sems[k]).wait()

    for k in range(_NBUF):
        issue(k, k)

    def ring(p, carry):
        r = p * _NBUF
        for k in range(_NBUF):
            wait(k)
            issue(r + k + _NBUF, k)
        return carry

    lax.fori_loop(0, (L - _NBUF) // _NBUF, ring, 0)
    for k in range(_NBUF):
        wait(k)

    def reduce_row(s, carry):
        a0 = acc_v[0, s, pl.ds(0, 16)]
        a1 = acc_v[0, s, pl.ds(16, 16)]
        for k in range(1, _NBUF):
            a0 = a0 + acc_v[k, s, pl.ds(0, 16)]
            a1 = a1 + acc_v[k, s, pl.ds(16, 16)]
        accf_v[s, pl.ds(0, 16)] = a0
        accf_v[s, pl.ds(16, 16)] = a1
        return carry

    lax.fori_loop(0, _SPW, reduce_row, 0)
    pltpu.sync_copy(accf_v, out_hbm.at[pl.ds(base, _SPW), :])


_pool = pl.kernel(
    _pool_body,
    out_type=jax.ShapeDtypeStruct((B, DIM), jnp.float32),
    mesh=plsc.VectorSubcoreMesh(core_axis_name="c", subcore_axis_name="s"),
    scratch_types=(
        [pltpu.VMEM((L, _SPW), jnp.int32),
         pltpu.VMEM((_NBUF, _SPW, DIM), jnp.float32),
         pltpu.VMEM((_SPW, DIM), jnp.float32)]
        + [pltpu.SemaphoreType.DMA] * _NBUF),
    compiler_params=pltpu.CompilerParams(use_tc_tiling_on_sc=False),
)


_VBLK = 2048
_NV = (VOCAB + _VBLK - 1) // _VBLK


def _mmT_body(p_ref, wt_ref, b_ref, o_ref):
    p = p_ref[...] * (1.0 / L)
    acc = lax.dot_general(
        wt_ref[...], p, (((0,), (1,)), ((), ())),
        preferred_element_type=jnp.float32)
    ones = jnp.ones((1, B), jnp.float32)
    bias = lax.dot_general(
        b_ref[...], ones, (((0,), (0,)), ((), ())),
        preferred_element_type=jnp.float32)
    o_ref[...] = acc + bias


def _matmul_t(pooled, WT, b2d):
    return pl.pallas_call(
        _mmT_body,
        grid=(_NV,),
        in_specs=[
            pl.BlockSpec((B, DIM), lambda i: (0, 0)),
            pl.BlockSpec((DIM, _VBLK), lambda i: (0, i)),
            pl.BlockSpec((1, _VBLK), lambda i: (0, i)),
        ],
        out_specs=pl.BlockSpec((_VBLK, B), lambda i: (i, 0)),
        out_shape=jax.ShapeDtypeStruct((VOCAB, B), jnp.float32),
    )(pooled, WT, b2d)


@jax.jit
def _impl(x, emb, W, b):
    sums = _pool(x.T, emb)
    logits_t = _matmul_t(sums, W.T, b.reshape(1, -1))
    return logits_t.T


def kernel(x, emb, W, b):
    return _impl(x, emb, W, b)


# speedup vs baseline: 1.4585x; 1.0024x over previous
import functools

import jax
import jax.numpy as jnp
from jax import lax
from jax.experimental import pallas as pl
from jax.experimental.pallas import tpu as pltpu
from jax.experimental.pallas import tpu_sc as plsc

VOCAB = 100000
DIM = 32
B = 1024
L = 200

_NC = 2
_NS = 16
_NW = _NC * _NS
_SPW = B // _NW


def _pool_body(x_hbm, emb_hbm, out_hbm, idx_v, rows_v, acc_v, sem_a, sem_b):
    wid = lax.axis_index("s") * _NC + lax.axis_index("c")
    base = wid * _SPW
    pltpu.sync_copy(x_hbm.at[pl.ds(base, _SPW), :], idx_v)

    def issue(s, buf, sem):
        pltpu.async_copy(
            emb_hbm.at[idx_v.at[s, pl.ds(0, 128)]],
            rows_v.at[buf, pl.ds(0, 128)], sem)
        pltpu.async_copy(
            emb_hbm.at[idx_v.at[s, pl.ds(128, L - 128)]],
            rows_v.at[buf, pl.ds(128, L - 128)], sem)

    def drain(buf, sem):
        pltpu.make_async_copy(
            emb_hbm.at[idx_v.at[0, pl.ds(0, 128)]],
            rows_v.at[buf, pl.ds(0, 128)], sem).wait()
        pltpu.make_async_copy(
            emb_hbm.at[idx_v.at[0, pl.ds(0, L - 128)]],
            rows_v.at[buf, pl.ds(128, L - 128)], sem).wait()

    def accum(buf, s):
        rows = rows_v.at[buf]

        def step(r, accs):
            a0, a1, a2, a3 = accs
            q = r * 4
            a0 = a0 + rows[q, pl.ds(0, 16)]
            a1 = a1 + rows[q, pl.ds(16, 16)]
            a2 = a2 + rows[q + 1, pl.ds(0, 16)]
            a3 = a3 + rows[q + 1, pl.ds(16, 16)]
            a0 = a0 + rows[q + 2, pl.ds(0, 16)]
            a1 = a1 + rows[q + 2, pl.ds(16, 16)]
            a2 = a2 + rows[q + 3, pl.ds(0, 16)]
            a3 = a3 + rows[q + 3, pl.ds(16, 16)]
            return a0, a1, a2, a3

        z = jnp.zeros((16,), jnp.float32)
        a0, a1, a2, a3 = lax.fori_loop(0, L // 4, step, (z, z, z, z))
        acc_v[s, pl.ds(0, 16)] = a0 + a2
        acc_v[s, pl.ds(16, 16)] = a1 + a3

    issue(0, 0, sem_a)

    def pair(p, carry):
        s = p * 2
        issue(s + 1, 1, sem_b)
        drain(0, sem_a)
        accum(0, s)

        @pl.when(p + 1 < _SPW // 2)
        def _():
            issue(s + 2, 0, sem_a)

        drain(1, sem_b)
        accum(1, s + 1)
        return carry

    lax.fori_loop(0, _SPW // 2, pair, 0)
    pltpu.sync_copy(acc_v, out_hbm.at[pl.ds(base, _SPW), :])


_pool = pl.kernel(
    _pool_body,
    out_type=jax.ShapeDtypeStruct((B, DIM), jnp.float32),
    mesh=plsc.VectorSubcoreMesh(core_axis_name="c", subcore_axis_name="s"),
    scratch_types=[
        pltpu.VMEM((_SPW, L), jnp.int32),
        pltpu.VMEM((2, L, DIM), jnp.float32),
        pltpu.VMEM((_SPW, DIM), jnp.float32),
        pltpu.SemaphoreType.DMA,
        pltpu.SemaphoreType.DMA,
    ],
    compiler_params=pltpu.CompilerParams(use_tc_tiling_on_sc=False),
)


_VBLK = 2048
_NV = (VOCAB + _VBLK - 1) // _VBLK


def _mmT_body(p_ref, wt_ref, b_ref, o_ref):
    p = p_ref[...] * (1.0 / L)
    acc = lax.dot_general(
        wt_ref[...], p, (((0,), (1,)), ((), ())),
        preferred_element_type=jnp.float32)
    ones = jnp.ones((1, B), jnp.float32)
    bias = lax.dot_general(
        b_ref[...], ones, (((0,), (0,)), ((), ())),
        preferred_element_type=jnp.float32)
    o_ref[...] = acc + bias


def _matmul_t(pooled, WT, b2d):
    return pl.pallas_call(
        _mmT_body,
        grid=(_NV,),
        in_specs=[
            pl.BlockSpec((B, DIM), lambda i: (0, 0)),
            pl.BlockSpec((DIM, _VBLK), lambda i: (0, i)),
            pl.BlockSpec((1, _VBLK), lambda i: (0, i)),
        ],
        out_specs=pl.BlockSpec((_VBLK, B), lambda i: (i, 0)),
        out_shape=jax.ShapeDtypeStruct((VOCAB, B), jnp.float32),
    )(pooled, WT, b2d)


@jax.jit
def _impl(x, emb, W, b):
    sums = _pool(x, emb)
    logits_t = _matmul_t(sums, W.T, b.reshape(1, -1))
    return logits_t.T


def kernel(x, emb, W, b):
    return _impl(x, emb, W, b)
